# Initial kernel scaffold; baseline (speedup 1.0000x reference)
import jax
import jax.numpy as jnp
from jax.experimental import pallas as pl


def kernel(features, roi):
    raise NotImplementedError("write your pallas kernel here")


# speedup vs baseline: 3.7360x; 3.7360x over previous
import functools

import jax
import jax.numpy as jnp
from jax import lax
from jax.experimental import pallas as pl
from jax.experimental.pallas import tpu as pltpu
from jax.experimental.pallas import tpu_sc as plsc

POOL = 7
NC, NS, L = 2, 16, 16
NW = NC * NS
CHUNK = 16


def _roi_pool_sc(table, roi_flat, *, n_pix, h_img, w_img, c_dim):
  n_chunks = n_pix // CHUNK
  assert n_pix % CHUNK == 0
  base_cnt, extra = divmod(n_chunks, NW)

  mesh = plsc.VectorSubcoreMesh(
      core_axis_name="c", subcore_axis_name="s", num_cores=NC,
      num_subcores=NS)

  @functools.partial(
      pl.kernel,
      out_type=jax.ShapeDtypeStruct((n_pix, c_dim), jnp.float32),
      mesh=mesh,
      scratch_types=[
          pltpu.VMEM(roi_flat.shape, jnp.int32),
          pltpu.VMEM((4, CHUNK, c_dim), jnp.float32),
          pltpu.VMEM((CHUNK, c_dim), jnp.float32),
          pltpu.VMEM((4, L), jnp.float32),
          pltpu.SemaphoreType.DMA,
      ],
      compiler_params=pltpu.CompilerParams(needs_layout_passes=False),
  )
  def k(table_hbm, roi_hbm, out_hbm, roi_v, rows_v, outb_v, wbuf_v, sem):
    wid = lax.axis_index("s") * NC + lax.axis_index("c")
    pltpu.sync_copy(roi_hbm, roi_v)
    cnt = base_cnt + jnp.where(wid < extra, 1, 0)

    lane = lax.iota(jnp.int32, L)
    pp = POOL * POOL

    def chunk_body(g, carry):
      c = wid + NW * g
      p = c * CHUNK + lane
      pf = p.astype(jnp.float32) + 0.5
      n = (pf * (1.0 / pp)).astype(jnp.int32)
      q = p - n * pp
      qf = q.astype(jnp.float32) + 0.5
      i = (qf * (1.0 / POOL)).astype(jnp.int32)
      j = q - i * POOL
      b = n * 4
      y1 = plsc.load_gather(roi_v, [b])
      x1 = plsc.load_gather(roi_v, [b + 1])
      y2 = plsc.load_gather(roi_v, [b + 2])
      x2 = plsc.load_gather(roi_v, [b + 3])
      h = jnp.maximum(x2 - x1, 1)
      w = jnp.maximum(y2 - y1, 1)
      rpos = i.astype(jnp.float32) * (h.astype(jnp.float32) * (1.0 / POOL))
      r0 = rpos.astype(jnp.int32)
      rf = rpos - r0.astype(jnp.float32)
      r1 = jnp.minimum(r0 + 1, h - 1)
      cpos = j.astype(jnp.float32) * (w.astype(jnp.float32) * (1.0 / POOL))
      c0 = cpos.astype(jnp.int32)
      cf = cpos - c0.astype(jnp.float32)
      c1 = jnp.minimum(c0 + 1, w - 1)
      row0 = x1 + r0
      row1 = x1 + r1
      col0 = y1 + c0
      col1 = y1 + c1
      cp0 = pltpu.async_copy(table_hbm.at[row0 * w_img + col0],
                             rows_v.at[0], sem)
      cp1 = pltpu.async_copy(table_hbm.at[row0 * w_img + col1],
                             rows_v.at[1], sem)
      cp2 = pltpu.async_copy(table_hbm.at[row1 * w_img + col0],
                             rows_v.at[2], sem)
      cp3 = pltpu.async_copy(table_hbm.at[row1 * w_img + col1],
                             rows_v.at[3], sem)
      wbuf_v[0, :] = (1.0 - rf) * (1.0 - cf)
      wbuf_v[1, :] = (1.0 - rf) * cf
      wbuf_v[2, :] = rf * (1.0 - cf)
      wbuf_v[3, :] = rf * cf
      cp0.wait()
      cp1.wait()
      cp2.wait()
      cp3.wait()

      def pix_body(px, carry2):
        pxv = jnp.full((L,), px, jnp.int32)
        w00 = plsc.load_gather(wbuf_v, [jnp.zeros((L,), jnp.int32), pxv])
        w01 = plsc.load_gather(wbuf_v, [jnp.full((L,), 1, jnp.int32), pxv])
        w10 = plsc.load_gather(wbuf_v, [jnp.full((L,), 2, jnp.int32), pxv])
        w11 = plsc.load_gather(wbuf_v, [jnp.full((L,), 3, jnp.int32), pxv])
        for cc in range(c_dim // L):
          sl = pl.ds(cc * L, L)
          acc = (rows_v[0, px, sl] * w00 + rows_v[1, px, sl] * w01 +
                 rows_v[2, px, sl] * w10 + rows_v[3, px, sl] * w11)
          outb_v[px, sl] = acc
        return carry2

      lax.fori_loop(0, CHUNK, pix_body, 0, unroll=False)
      pltpu.sync_copy(outb_v, out_hbm.at[pl.ds(c * CHUNK, CHUNK)])
      return carry

    lax.fori_loop(0, cnt, chunk_body, 0, unroll=False)

  return k(table, roi_flat)


def kernel(features, roi):
  b, h_img, w_img, c_dim = features.shape
  n_roi = roi.shape[1]
  assert b == 1
  table = features.reshape(h_img * w_img, c_dim)
  roi_flat = roi.astype(jnp.int32).reshape(-1)
  n_pix = n_roi * POOL * POOL
  out = _roi_pool_sc(table, roi_flat, n_pix=n_pix, h_img=h_img,
                     w_img=w_img, c_dim=c_dim)
  return out.reshape(n_roi, POOL, POOL, c_dim)


# speedup vs baseline: 4.9679x; 1.3297x over previous
import functools

import jax
import jax.numpy as jnp
from jax import lax
from jax.experimental import pallas as pl
from jax.experimental.pallas import tpu as pltpu
from jax.experimental.pallas import tpu_sc as plsc

POOL = 7
NC, NS, L = 2, 16, 16
NW = NC * NS
CHUNK = 16
NB = 4


def _roi_pool_sc(table, roi_flat, *, n_pix, h_img, w_img, c_dim):
  n_chunks = n_pix // CHUNK
  assert n_pix % CHUNK == 0
  base_cnt, extra = divmod(n_chunks, NW)
  rounds = -(-(base_cnt + (1 if extra else 0)) // NB)

  mesh = plsc.VectorSubcoreMesh(
      core_axis_name="c", subcore_axis_name="s", num_cores=NC,
      num_subcores=NS)

  @functools.partial(
      pl.kernel,
      out_type=jax.ShapeDtypeStruct((n_pix, c_dim), jnp.float32),
      mesh=mesh,
      scratch_types=[
          pltpu.VMEM(roi_flat.shape, jnp.int32),
          pltpu.VMEM((NB, 4 * CHUNK), jnp.int32),
          pltpu.VMEM((NB, 4 * CHUNK, c_dim), jnp.float32),
          pltpu.VMEM((NB, CHUNK, c_dim), jnp.float32),
          pltpu.VMEM((NB, 4, L), jnp.float32),
      ] + [pltpu.SemaphoreType.DMA] * (2 * NB),
      compiler_params=pltpu.CompilerParams(needs_layout_passes=False),
  )
  def k(table_hbm, roi_hbm, out_hbm, roi_v, idx_v, rows_v, outb_v, wbuf_v,
        *sems):
    gsem = sems[:NB]
    osem = sems[NB:]
    wid = lax.axis_index("s") * NC + lax.axis_index("c")
    pltpu.sync_copy(roi_hbm, roi_v)
    cnt = base_cnt + jnp.where(wid < extra, 1, 0)

    lane = lax.iota(jnp.int32, L)
    pp = POOL * POOL

    def stage_chunk(t, b):
      c = wid + NW * t
      p = c * CHUNK + lane
      pf = p.astype(jnp.float32) + 0.5
      n = (pf * (1.0 / pp)).astype(jnp.int32)
      q = p - n * pp
      qf = q.astype(jnp.float32) + 0.5
      i = (qf * (1.0 / POOL)).astype(jnp.int32)
      j = q - i * POOL
      b4 = n * 4
      y1 = plsc.load_gather(roi_v, [b4])
      x1 = plsc.load_gather(roi_v, [b4 + 1])
      y2 = plsc.load_gather(roi_v, [b4 + 2])
      x2 = plsc.load_gather(roi_v, [b4 + 3])
      h = jnp.maximum(x2 - x1, 1)
      w = jnp.maximum(y2 - y1, 1)
      rpos = i.astype(jnp.float32) * (h.astype(jnp.float32) * (1.0 / POOL))
      r0 = rpos.astype(jnp.int32)
      rf = rpos - r0.astype(jnp.float32)
      r1 = jnp.minimum(r0 + 1, h - 1)
      cpos = j.astype(jnp.float32) * (w.astype(jnp.float32) * (1.0 / POOL))
      c0 = cpos.astype(jnp.int32)
      cf = cpos - c0.astype(jnp.float32)
      c1 = jnp.minimum(c0 + 1, w - 1)
      base00 = (x1 + r0) * w_img + y1
      base1 = (x1 + r1) * w_img + y1
      idx_v[b, pl.ds(0, L)] = base00 + c0
      idx_v[b, pl.ds(L, L)] = base00 + c1
      idx_v[b, pl.ds(2 * L, L)] = base1 + c0
      idx_v[b, pl.ds(3 * L, L)] = base1 + c1
      wbuf_v[b, 0, :] = (1.0 - rf) * (1.0 - cf)
      wbuf_v[b, 1, :] = (1.0 - rf) * cf
      wbuf_v[b, 2, :] = rf * (1.0 - cf)
      wbuf_v[b, 3, :] = rf * cf
      pltpu.async_copy(table_hbm.at[idx_v.at[b]], rows_v.at[b], gsem[b])

    def drain_gather(b):
      pltpu.make_async_copy(table_hbm.at[pl.ds(0, 4 * CHUNK)], rows_v.at[b],
                            gsem[b]).wait()

    def drain_write(b):
      pltpu.make_async_copy(outb_v.at[b], out_hbm.at[pl.ds(0, CHUNK)],
                            osem[b]).wait()

    for b in range(NB):
      @pl.when(b < cnt)
      def _(b=b):
        stage_chunk(jnp.int32(b), b)

    def round_body(r, carry):
      for b in range(NB):
        t = r * NB + b

        @pl.when(t < cnt)
        def _(t=t, b=b):
          drain_gather(b)

          @pl.when(r > 0)
          def _():
            drain_write(b)

          def pix_body(px, carry2):
            pxv = jnp.full((L,), px, jnp.int32)
            bv = jnp.full((L,), b, jnp.int32)
            w00 = plsc.load_gather(wbuf_v, [bv, jnp.full((L,), 0, jnp.int32),
                                            pxv])
            w01 = plsc.load_gather(wbuf_v, [bv, jnp.full((L,), 1, jnp.int32),
                                            pxv])
            w10 = plsc.load_gather(wbuf_v, [bv, jnp.full((L,), 2, jnp.int32),
                                            pxv])
            w11 = plsc.load_gather(wbuf_v, [bv, jnp.full((L,), 3, jnp.int32),
                                            pxv])
            for cc in range(c_dim // L):
              sl = pl.ds(cc * L, L)
              acc = (rows_v[b, px, sl] * w00 +
                     rows_v[b, L + px, sl] * w01 +
                     rows_v[b, 2 * L + px, sl] * w10 +
                     rows_v[b, 3 * L + px, sl] * w11)
              outb_v[b, px, sl] = acc
            return carry2

          lax.fori_loop(0, CHUNK, pix_body, 0, unroll=False)
          c = wid + NW * t
          pltpu.async_copy(outb_v.at[b], out_hbm.at[pl.ds(c * CHUNK, CHUNK)],
                           osem[b])
          t2 = t + NB

          @pl.when(t2 < cnt)
          def _():
            stage_chunk(t2, b)

      return carry

    lax.fori_loop(0, rounds, round_body, 0, unroll=False)
    for b in range(NB):
      drain_write(b)

  return k(table, roi_flat)


def kernel(features, roi):
  b, h_img, w_img, c_dim = features.shape
  n_roi = roi.shape[1]
  assert b == 1
  table = features.reshape(h_img * w_img, c_dim)
  roi_flat = roi.astype(jnp.int32).reshape(-1)
  n_pix = n_roi * POOL * POOL
  out = _roi_pool_sc(table, roi_flat, n_pix=n_pix, h_img=h_img,
                     w_img=w_img, c_dim=c_dim)
  return out.reshape(n_roi, POOL, POOL, c_dim)
